# baseline (speedup 1.0000x reference)
import functools

import jax
import jax.numpy as jnp
from jax import lax
from jax.experimental import pallas as pl
from jax.experimental.pallas import tpu as pltpu

_TOP_RATIO = 0.3
_TOP_WEIGHT = 1.0
_IGNORE_INDEX = -1

_VB = 2000
_CH = 500
_NW = 2


def _lse_one(gt_row, pred_ref, lsep_ref, gathp_ref, block_idx):
    vb, n = pred_ref.shape
    nch = vb // _CH

    m = jnp.max(pred_ref[pl.ds(0, _CH), :], axis=0, keepdims=True)
    for ch in range(1, nch):
        x = pred_ref[pl.ds(ch * _CH, _CH), :]
        m = jnp.maximum(m, jnp.max(x, axis=0, keepdims=True))

    target = gt_row - block_idx * vb
    s = jnp.zeros((1, n), jnp.float32)
    g = jnp.zeros((1, n), jnp.float32)
    for ch in range(nch):
        x = pred_ref[pl.ds(ch * _CH, _CH), :]
        s = s + jnp.sum(jnp.exp(x - m), axis=0, keepdims=True)
        rows = lax.broadcasted_iota(jnp.int32, (_CH, n), 0) + ch * _CH
        g = g + jnp.sum(jnp.where(rows == target, x, jnp.float32(0.0)),
                        axis=0, keepdims=True)

    lsep_ref[...] = (m + jnp.log(s))[None]
    gathp_ref[...] = g[None]


def _lse_body(nb, gt_ref, *refs):
    j = pl.program_id(0)
    gt_row = gt_ref[...]
    preds = refs[:_NW]
    outs = refs[_NW:]
    for w in range(_NW):
        _lse_one(gt_row, preds[w], outs[2 * w], outs[2 * w + 1], w * nb + j)


def _lse_parts(pred_t, gt_row):
    c, n = pred_t.shape
    nb = c // (_VB * _NW)
    in_specs = [pl.BlockSpec((1, n), lambda j: (0, 0))] + [
        pl.BlockSpec((_VB, n), functools.partial(lambda w, j: (nb * w + j, 0), w))
        for w in range(_NW)
    ]
    out_specs = [pl.BlockSpec((1, 1, n), lambda j: (j, 0, 0))] * (2 * _NW)
    outs = pl.pallas_call(
        functools.partial(_lse_body, nb),
        grid=(nb,),
        in_specs=in_specs,
        out_specs=out_specs,
        out_shape=[jax.ShapeDtypeStruct((nb, 1, n), jnp.float32)] * (2 * _NW),
    )(gt_row, *([pred_t] * _NW))
    lsep = jnp.concatenate(outs[0::2], axis=0)
    gathp = jnp.concatenate(outs[1::2], axis=0)
    return lsep, gathp


def _final_body(n, k, lsep_ref, gathp_ref, gt_ref, out_ref):
    lsep = lsep_ref[...]
    m = jnp.max(lsep, axis=0, keepdims=True)
    s = jnp.sum(jnp.exp(lsep - m), axis=0, keepdims=True)
    lse = m + jnp.log(s)
    gat = jnp.sum(gathp_ref[...], axis=0, keepdims=True)

    nll = lse - gat
    valid = gt_ref[...] != _IGNORE_INDEX
    loss = jnp.where(valid, nll, jnp.float32(0.0))
    total = jnp.sum(loss)

    bb = lax.bitcast_convert_type(loss, jnp.int32)
    skey = bb ^ (lax.shift_right_arithmetic(bb, 31) & jnp.int32(0x7FFFFFFF))
    int_min = jnp.int32(-2147483648)

    def step(i, p):
        cand = p | lax.shift_left(jnp.int32(1), 31 - i)
        cnt = jnp.sum((skey >= (cand ^ int_min)).astype(jnp.int32))
        return jnp.where(cnt >= k, cand, p)

    p = lax.fori_loop(0, 32, step, jnp.int32(0))
    skey_th = p ^ int_min
    cnt_gt = jnp.sum((skey > skey_th).astype(jnp.int32))
    sum_gt = jnp.sum(jnp.where(skey > skey_th, loss, jnp.float32(0.0)))
    bits_th = skey_th ^ (lax.shift_right_arithmetic(skey_th, 31) & jnp.int32(0x7FFFFFFF))
    f_th = lax.bitcast_convert_type(bits_th, jnp.float32)
    topk_sum = sum_gt + (k - cnt_gt).astype(jnp.float32) * f_th

    out = total / jnp.float32(n) + jnp.float32(_TOP_WEIGHT) * topk_sum / jnp.float32(k)
    out_ref[...] = jnp.full((1, 1), out, jnp.float32)


def _finalize(lsep, gathp, gt_row, n, k):
    return pl.pallas_call(
        functools.partial(_final_body, n, k),
        out_shape=jax.ShapeDtypeStruct((1, 1), jnp.float32),
    )(lsep, gathp, gt_row)


def kernel(pred, gt):
    n, c = pred.shape
    k = max(int(_TOP_RATIO * n), 1)
    gt_row = gt.reshape(1, n)
    lsep, gathp = _lse_parts(pred.T, gt_row)
    nb = c // _VB
    out = _finalize(lsep.reshape(nb, n), gathp.reshape(nb, n), gt_row, n, k)
    return out[0, 0]


# speedup vs baseline: 1.0287x; 1.0287x over previous
import functools

import jax
import jax.numpy as jnp
from jax import lax
from jax.experimental import pallas as pl
from jax.experimental.pallas import tpu as pltpu

_TOP_RATIO = 0.3
_TOP_WEIGHT = 1.0
_IGNORE_INDEX = -1

_VB = 2000
_CH = 500
_NW = 1


def _lse_one(gt_row, pred_ref, lsep_ref, gathp_ref, block_idx):
    vb, n = pred_ref.shape
    nch = vb // _CH

    m = jnp.max(pred_ref[pl.ds(0, _CH), :], axis=0, keepdims=True)
    for ch in range(1, nch):
        x = pred_ref[pl.ds(ch * _CH, _CH), :]
        m = jnp.maximum(m, jnp.max(x, axis=0, keepdims=True))

    target = gt_row - block_idx * vb
    s = jnp.zeros((1, n), jnp.float32)
    g = jnp.zeros((1, n), jnp.float32)
    for ch in range(nch):
        x = pred_ref[pl.ds(ch * _CH, _CH), :]
        s = s + jnp.sum(jnp.exp(x - m), axis=0, keepdims=True)
        rows = lax.broadcasted_iota(jnp.int32, (_CH, n), 0) + ch * _CH
        g = g + jnp.sum(jnp.where(rows == target, x, jnp.float32(0.0)),
                        axis=0, keepdims=True)

    lsep_ref[...] = (m + jnp.log(s))[None]
    gathp_ref[...] = g[None]


def _lse_body(nb, gt_ref, *refs):
    j = pl.program_id(0)
    gt_row = gt_ref[...]
    preds = refs[:_NW]
    outs = refs[_NW:]
    for w in range(_NW):
        _lse_one(gt_row, preds[w], outs[2 * w], outs[2 * w + 1], w * nb + j)


def _lse_parts(pred_t, gt_row):
    c, n = pred_t.shape
    nb = c // (_VB * _NW)
    in_specs = [pl.BlockSpec((1, n), lambda j: (0, 0))] + [
        pl.BlockSpec((_VB, n), functools.partial(lambda w, j: (nb * w + j, 0), w))
        for w in range(_NW)
    ]
    out_specs = [pl.BlockSpec((1, 1, n), lambda j: (j, 0, 0))] * (2 * _NW)
    outs = pl.pallas_call(
        functools.partial(_lse_body, nb),
        grid=(nb,),
        in_specs=in_specs,
        out_specs=out_specs,
        out_shape=[jax.ShapeDtypeStruct((nb, 1, n), jnp.float32)] * (2 * _NW),
    )(gt_row, *([pred_t] * _NW))
    lsep = jnp.concatenate(outs[0::2], axis=0)
    gathp = jnp.concatenate(outs[1::2], axis=0)
    return lsep, gathp


def _final_body(n, k, lsep_ref, gathp_ref, gt_ref, out_ref):
    lsep = lsep_ref[...]
    m = jnp.max(lsep, axis=0, keepdims=True)
    s = jnp.sum(jnp.exp(lsep - m), axis=0, keepdims=True)
    lse = m + jnp.log(s)
    gat = jnp.sum(gathp_ref[...], axis=0, keepdims=True)

    nll = lse - gat
    valid = gt_ref[...] != _IGNORE_INDEX
    loss = jnp.where(valid, nll, jnp.float32(0.0))
    total = jnp.sum(loss)

    bb = lax.bitcast_convert_type(loss, jnp.int32)
    skey = bb ^ (lax.shift_right_arithmetic(bb, 31) & jnp.int32(0x7FFFFFFF))
    int_min = jnp.int32(-2147483648)

    def step(i, p):
        cand = p | lax.shift_left(jnp.int32(1), 31 - i)
        cnt = jnp.sum((skey >= (cand ^ int_min)).astype(jnp.int32))
        return jnp.where(cnt >= k, cand, p)

    p = lax.fori_loop(0, 32, step, jnp.int32(0))
    skey_th = p ^ int_min
    cnt_gt = jnp.sum((skey > skey_th).astype(jnp.int32))
    sum_gt = jnp.sum(jnp.where(skey > skey_th, loss, jnp.float32(0.0)))
    bits_th = skey_th ^ (lax.shift_right_arithmetic(skey_th, 31) & jnp.int32(0x7FFFFFFF))
    f_th = lax.bitcast_convert_type(bits_th, jnp.float32)
    topk_sum = sum_gt + (k - cnt_gt).astype(jnp.float32) * f_th

    out = total / jnp.float32(n) + jnp.float32(_TOP_WEIGHT) * topk_sum / jnp.float32(k)
    out_ref[...] = jnp.full((1, 1), out, jnp.float32)


def _finalize(lsep, gathp, gt_row, n, k):
    return pl.pallas_call(
        functools.partial(_final_body, n, k),
        out_shape=jax.ShapeDtypeStruct((1, 1), jnp.float32),
    )(lsep, gathp, gt_row)


def kernel(pred, gt):
    n, c = pred.shape
    k = max(int(_TOP_RATIO * n), 1)
    gt_row = gt.reshape(1, n)
    lsep, gathp = _lse_parts(pred.T, gt_row)
    nb = c // _VB
    out = _finalize(lsep.reshape(nb, n), gathp.reshape(nb, n), gt_row, n, k)
    return out[0, 0]
